# baseline (speedup 1.0000x reference)
import functools

import jax
import jax.numpy as jnp
from jax import lax
from jax.experimental import pallas as pl
from jax.experimental.pallas import tpu as pltpu
from jax.experimental.pallas import tpu_sc as plsc

_INFEATURES = 512
_CAPACITY = 16384
_EPS = 1e-8

_NW = 32
_SC_RPW = 128
_SC_ROWS = _NW * _SC_RPW
_TC_ROWS = _CAPACITY - _SC_ROWS
_CHUNK = 32
_NCHUNK = _SC_RPW // _CHUNK
_NSLICE = _INFEATURES // 16
_RB = 4
_TC_BLOCK = 1024
_TC_STEPS = _TC_ROWS // (2 * _TC_BLOCK)


def _phase1_body(x_hbm, mem_hbm, d_out, n_out, i_out,
                 xv, b0, b1, pd, pn, od, on, oi, sem0, sem1):
    wid = lax.axis_index("c") * 16 + lax.axis_index("s")
    base_row = wid * _SC_RPW

    pltpu.sync_copy(x_hbm, xv)
    xs = [xv[pl.ds(s * 16, 16)] for s in range(_NSLICE)]

    bufs = (b0, b1)
    sems = (sem0, sem1)

    def start(c, k):
        pltpu.async_copy(mem_hbm.at[pl.ds(base_row + c * _CHUNK, _CHUNK)],
                         bufs[k], sems[k])

    def wait(k):
        pltpu.make_async_copy(mem_hbm.at[pl.ds(0, _CHUNK)],
                              bufs[k], sems[k]).wait()

    def process(c, k):
        buf = bufs[k]

        def row_body(rb, _):
            r = rb * _RB
            accs = []
            for q in range(_RB):
                m = buf[r + q, pl.ds(0, 16)]
                accs.append([m * xs[0], m * m])
            for s in range(1, _NSLICE):
                xv_s = xs[s]
                for q in range(_RB):
                    m = buf[r + q, pl.ds(s * 16, 16)]
                    accs[q][0] = accs[q][0] + m * xv_s
                    accs[q][1] = accs[q][1] + m * m
            for q in range(_RB):
                pd[pl.ds((c * _CHUNK + r + q) * 16, 16)] = accs[q][0]
                pn[pl.ds((c * _CHUNK + r + q) * 16, 16)] = accs[q][1]
            return 0

        lax.fori_loop(0, _CHUNK // _RB, row_body, 0)

    start(0, 0)
    start(1, 1)

    def chunk_body(j, _):
        c = j * 2
        wait(0)
        process(c, 0)
        start(c + 2, 0)
        wait(1)
        process(c + 1, 1)
        start(c + 3, 1)
        return 0

    lax.fori_loop(0, _NCHUNK // 2 - 1, chunk_body, 0)
    wait(0)
    process(_NCHUNK - 2, 0)
    wait(1)
    process(_NCHUNK - 1, 1)

    def red_body(rb, carry):
        r4 = rb * 4
        ds_ = [jnp.sum(pd[pl.ds((r4 + q) * 16, 16)]) for q in range(4)]
        ns_ = [jnp.sum(pn[pl.ds((r4 + q) * 16, 16)]) for q in range(4)]
        for q in range(4):
            bnum, bden, bd, bn, bi = carry
            d = ds_[q]
            n = ns_[q]
            num = d * jnp.abs(d)
            den = jnp.maximum(n, 1e-16)
            pred = num * bden > bnum * den
            carry = (jnp.where(pred, num, bnum),
                     jnp.where(pred, den, bden),
                     jnp.where(pred, d, bd),
                     jnp.where(pred, n, bn),
                     jnp.where(pred, base_row + r4 + q, bi))
        return carry

    init = (jnp.float32(-3.4e38), jnp.float32(1.0), jnp.float32(0.0),
            jnp.float32(1.0), jnp.int32(0))
    _, _, bd, bn, bi = lax.fori_loop(0, _SC_RPW // 4, red_body, init)

    od[...] = jnp.full((16,), bd, jnp.float32)
    on[...] = jnp.full((16,), bn, jnp.float32)
    oi[...] = jnp.full((16,), bi, jnp.int32)
    pltpu.sync_copy(od, d_out.at[pl.ds(wid * 16, 16)])
    pltpu.sync_copy(on, n_out.at[pl.ds(wid * 16, 16)])
    pltpu.sync_copy(oi, i_out.at[pl.ds(wid * 16, 16)])


@functools.lru_cache(maxsize=None)
def _build_phase1():
    mesh = plsc.VectorSubcoreMesh(core_axis_name="c", subcore_axis_name="s")
    return functools.partial(
        pl.kernel,
        out_type=[
            jax.ShapeDtypeStruct((_NW * 16,), jnp.float32),
            jax.ShapeDtypeStruct((_NW * 16,), jnp.float32),
            jax.ShapeDtypeStruct((_NW * 16,), jnp.int32),
        ],
        mesh=mesh,
        compiler_params=pltpu.CompilerParams(needs_layout_passes=False),
        scratch_types=[
            pltpu.VMEM((_INFEATURES,), jnp.float32),
            pltpu.VMEM((_CHUNK, _INFEATURES), jnp.float32),
            pltpu.VMEM((_CHUNK, _INFEATURES), jnp.float32),
            pltpu.VMEM((_SC_RPW * 16,), jnp.float32),
            pltpu.VMEM((_SC_RPW * 16,), jnp.float32),
            pltpu.VMEM((16,), jnp.float32),
            pltpu.VMEM((16,), jnp.float32),
            pltpu.VMEM((16,), jnp.int32),
            pltpu.SemaphoreType.DMA,
            pltpu.SemaphoreType.DMA,
        ],
    )(_phase1_body)


def _tc_scan_body(x_ref, ma_ref, mb_ref, bd_ref, bn_ref, bi_ref, sm_f, sm_i):
    step = pl.program_id(0)
    xr = x_ref[...].reshape(1, _INFEATURES)
    ones = jnp.ones((1, _INFEATURES), jnp.float32)

    @pl.when(step == 0)
    def _():
        sm_f[0] = jnp.float32(-3.4e38)
        sm_f[1] = jnp.float32(0.0)
        sm_f[2] = jnp.float32(1.0)
        sm_i[0] = jnp.int32(2**31 - 1)

    for h, m_ref in enumerate((ma_ref, mb_ref)):
        m = m_ref[...]
        d = lax.dot_general(xr, m, (((1,), (1,)), ((), ())),
                            preferred_element_type=jnp.float32)
        n = lax.dot_general(ones, m * m, (((1,), (1,)), ((), ())),
                            preferred_element_type=jnp.float32)
        g = d * jnp.abs(d) / jnp.maximum(n, 1e-16)
        ti = (_SC_ROWS + (step * 2 + h) * _TC_BLOCK
              + lax.broadcasted_iota(jnp.int32, (1, _TC_BLOCK), 1))
        gm = jnp.max(g)
        bi = jnp.min(jnp.where(g == gm, ti, jnp.int32(2**31 - 1)))
        sel = ti == bi
        bd = jnp.max(jnp.where(sel, d, -3.4e38))
        bn = jnp.max(jnp.where(sel, n, -3.4e38))

        @pl.when(gm > sm_f[0])
        def _():
            sm_f[0] = gm
            sm_f[1] = bd
            sm_f[2] = bn
            sm_i[0] = bi

    @pl.when(step == _TC_STEPS - 1)
    def _():
        bd_ref[...] = jnp.full((1, 128), sm_f[1], jnp.float32)
        bn_ref[...] = jnp.full((1, 128), sm_f[2], jnp.float32)
        bi_ref[...] = jnp.full((1, 128), sm_i[0], jnp.int32)


_tc_scan = pl.pallas_call(
    _tc_scan_body,
    grid=(_TC_STEPS,),
    in_specs=[
        pl.BlockSpec((_INFEATURES,), lambda i: (0,)),
        pl.BlockSpec((_TC_BLOCK, _INFEATURES),
                     lambda i: (i * 2 + _SC_ROWS // _TC_BLOCK, 0)),
        pl.BlockSpec((_TC_BLOCK, _INFEATURES),
                     lambda i: (i * 2 + 1 + _SC_ROWS // _TC_BLOCK, 0)),
    ],
    out_specs=[
        pl.BlockSpec((1, 128), lambda i: (0, 0)),
        pl.BlockSpec((1, 128), lambda i: (0, 0)),
        pl.BlockSpec((1, 128), lambda i: (0, 0)),
    ],
    out_shape=[
        jax.ShapeDtypeStruct((1, 128), jnp.float32),
        jax.ShapeDtypeStruct((1, 128), jnp.float32),
        jax.ShapeDtypeStruct((1, 128), jnp.int32),
    ],
    scratch_shapes=[
        pltpu.SMEM((4,), jnp.float32),
        pltpu.SMEM((2,), jnp.int32),
    ],
)


def _merge_body(x_ref, sd_ref, sn_ref, si_ref, td_ref, tn_ref, ti_ref,
                out_ref):
    x = x_ref[...]
    xx = jnp.sum(x * x)
    big_i = jnp.int32(2**31 - 1)

    sd = sd_ref[...]
    sn = sn_ref[...]
    si = si_ref[...]
    gs = sd * jnp.abs(sd) / jnp.maximum(sn, 1e-16)

    td = jnp.max(td_ref[...])
    tn = jnp.max(tn_ref[...])
    ti = jnp.max(ti_ref[...])
    gt = td * jnp.abs(td) / jnp.maximum(tn, 1e-16)

    gm = jnp.maximum(jnp.max(gs), gt)
    bi = jnp.minimum(jnp.min(jnp.where(gs == gm, si, big_i)),
                     jnp.where(gt == gm, ti, big_i))
    in_sc = jnp.min(jnp.where(gs == gm, si, big_i)) <= bi
    bd = jnp.where(bi == ti, td, jnp.max(jnp.where(si == bi, sd, -3.4e38)))
    bn = jnp.where(bi == ti, tn, jnp.max(jnp.where(si == bi, sn, -3.4e38)))
    bd = jnp.where(in_sc, jnp.max(jnp.where(si == bi, sd, -3.4e38)), bd)
    bn = jnp.where(in_sc, jnp.max(jnp.where(si == bi, sn, -3.4e38)), bn)
    val = bd / (jnp.maximum(jnp.sqrt(bn), _EPS) * jnp.maximum(jnp.sqrt(xx), _EPS))
    fi = lax.broadcasted_iota(jnp.int32, (_CAPACITY,), 0)
    out_ref[...] = jnp.where(fi == bi, val, 0.0)


_merge = pl.pallas_call(
    _merge_body,
    out_shape=jax.ShapeDtypeStruct((_CAPACITY,), jnp.float32),
)


@jax.jit
def kernel(x, memory):
    sd, sn, si = _build_phase1()(x, memory)
    td, tn, ti = _tc_scan(x, memory, memory)
    return _merge(x, sd, sn, si, td, tn, ti)


# speedup vs baseline: 1.0195x; 1.0195x over previous
import functools

import jax
import jax.numpy as jnp
from jax import lax
from jax.experimental import pallas as pl
from jax.experimental.pallas import tpu as pltpu
from jax.experimental.pallas import tpu_sc as plsc

_INFEATURES = 512
_CAPACITY = 16384
_EPS = 1e-8

_NW = 32
_SC_RPW = 128
_SC_ROWS = _NW * _SC_RPW
_TC_ROWS = _CAPACITY - _SC_ROWS
_CHUNK = 32
_NCHUNK = _SC_RPW // _CHUNK
_NSLICE = _INFEATURES // 16
_RB = 4
_TC_BLOCK = 2048
_TC_STEPS = _TC_ROWS // _TC_BLOCK


def _phase1_body(x_hbm, mem_hbm, d_out, n_out, i_out,
                 xv, b0, b1, pd, pn, od, on, oi, sem0, sem1):
    wid = lax.axis_index("c") * 16 + lax.axis_index("s")
    base_row = wid * _SC_RPW

    pltpu.sync_copy(x_hbm, xv)
    xs = [xv[pl.ds(s * 16, 16)] for s in range(_NSLICE)]

    bufs = (b0, b1)
    sems = (sem0, sem1)

    def start(c, k):
        pltpu.async_copy(mem_hbm.at[pl.ds(base_row + c * _CHUNK, _CHUNK)],
                         bufs[k], sems[k])

    def wait(k):
        pltpu.make_async_copy(mem_hbm.at[pl.ds(0, _CHUNK)],
                              bufs[k], sems[k]).wait()

    def process(c, k):
        buf = bufs[k]

        def row_body(rb, _):
            r = rb * _RB
            accs = []
            for q in range(_RB):
                m = buf[r + q, pl.ds(0, 16)]
                accs.append([m * xs[0], m * m])
            for s in range(1, _NSLICE):
                xv_s = xs[s]
                for q in range(_RB):
                    m = buf[r + q, pl.ds(s * 16, 16)]
                    accs[q][0] = accs[q][0] + m * xv_s
                    accs[q][1] = accs[q][1] + m * m
            for q in range(_RB):
                pd[pl.ds((c * _CHUNK + r + q) * 16, 16)] = accs[q][0]
                pn[pl.ds((c * _CHUNK + r + q) * 16, 16)] = accs[q][1]
            return 0

        lax.fori_loop(0, _CHUNK // _RB, row_body, 0)

    start(0, 0)
    start(1, 1)

    def chunk_body(j, _):
        c = j * 2
        wait(0)
        process(c, 0)

        @pl.when(j < _NCHUNK // 2 - 1)
        def _():
            start(c + 2, 0)

        wait(1)
        process(c + 1, 1)

        @pl.when(j < _NCHUNK // 2 - 1)
        def _():
            start(c + 3, 1)

        return 0

    lax.fori_loop(0, _NCHUNK // 2, chunk_body, 0)

    def red_body(rb, carry):
        r4 = rb * 4
        ds_ = [jnp.sum(pd[pl.ds((r4 + q) * 16, 16)]) for q in range(4)]
        ns_ = [jnp.sum(pn[pl.ds((r4 + q) * 16, 16)]) for q in range(4)]
        for q in range(4):
            bnum, bden, bd, bn, bi = carry
            d = ds_[q]
            n = ns_[q]
            num = d * jnp.abs(d)
            den = jnp.maximum(n, 1e-16)
            pred = num * bden > bnum * den
            carry = (jnp.where(pred, num, bnum),
                     jnp.where(pred, den, bden),
                     jnp.where(pred, d, bd),
                     jnp.where(pred, n, bn),
                     jnp.where(pred, base_row + r4 + q, bi))
        return carry

    init = (jnp.float32(-3.4e38), jnp.float32(1.0), jnp.float32(0.0),
            jnp.float32(1.0), jnp.int32(0))
    _, _, bd, bn, bi = lax.fori_loop(0, _SC_RPW // 4, red_body, init)

    od[...] = jnp.full((16,), bd, jnp.float32)
    on[...] = jnp.full((16,), bn, jnp.float32)
    oi[...] = jnp.full((16,), bi, jnp.int32)
    pltpu.sync_copy(od, d_out.at[pl.ds(wid * 16, 16)])
    pltpu.sync_copy(on, n_out.at[pl.ds(wid * 16, 16)])
    pltpu.sync_copy(oi, i_out.at[pl.ds(wid * 16, 16)])


@functools.lru_cache(maxsize=None)
def _build_phase1():
    mesh = plsc.VectorSubcoreMesh(core_axis_name="c", subcore_axis_name="s")
    return functools.partial(
        pl.kernel,
        out_type=[
            jax.ShapeDtypeStruct((_NW * 16,), jnp.float32),
            jax.ShapeDtypeStruct((_NW * 16,), jnp.float32),
            jax.ShapeDtypeStruct((_NW * 16,), jnp.int32),
        ],
        mesh=mesh,
        compiler_params=pltpu.CompilerParams(needs_layout_passes=False),
        scratch_types=[
            pltpu.VMEM((_INFEATURES,), jnp.float32),
            pltpu.VMEM((_CHUNK, _INFEATURES), jnp.float32),
            pltpu.VMEM((_CHUNK, _INFEATURES), jnp.float32),
            pltpu.VMEM((_SC_RPW * 16,), jnp.float32),
            pltpu.VMEM((_SC_RPW * 16,), jnp.float32),
            pltpu.VMEM((16,), jnp.float32),
            pltpu.VMEM((16,), jnp.float32),
            pltpu.VMEM((16,), jnp.int32),
            pltpu.SemaphoreType.DMA,
            pltpu.SemaphoreType.DMA,
        ],
    )(_phase1_body)


def _tc_scan_body(x_ref, m_ref, bd_ref, bn_ref, bi_ref, sm_f, sm_i):
    step = pl.program_id(0)
    xr = x_ref[...].reshape(1, _INFEATURES)
    ones = jnp.ones((1, _INFEATURES), jnp.float32)

    @pl.when(step == 0)
    def _():
        sm_f[0] = jnp.float32(-3.4e38)
        sm_f[1] = jnp.float32(0.0)
        sm_f[2] = jnp.float32(1.0)
        sm_i[0] = jnp.int32(2**31 - 1)

    m = m_ref[...]
    d = lax.dot_general(xr, m, (((1,), (1,)), ((), ())),
                        preferred_element_type=jnp.float32)
    n = lax.dot_general(ones, m * m, (((1,), (1,)), ((), ())),
                        preferred_element_type=jnp.float32)
    g = d * jnp.abs(d) / jnp.maximum(n, 1e-16)
    ti = (_SC_ROWS + step * _TC_BLOCK
          + lax.broadcasted_iota(jnp.int32, (1, _TC_BLOCK), 1))
    gm = jnp.max(g)
    bi = jnp.min(jnp.where(g == gm, ti, jnp.int32(2**31 - 1)))
    sel = ti == bi
    bd = jnp.max(jnp.where(sel, d, -3.4e38))
    bn = jnp.max(jnp.where(sel, n, -3.4e38))

    @pl.when(gm > sm_f[0])
    def _():
        sm_f[0] = gm
        sm_f[1] = bd
        sm_f[2] = bn
        sm_i[0] = bi

    @pl.when(step == _TC_STEPS - 1)
    def _():
        bd_ref[...] = jnp.full((1, 128), sm_f[1], jnp.float32)
        bn_ref[...] = jnp.full((1, 128), sm_f[2], jnp.float32)
        bi_ref[...] = jnp.full((1, 128), sm_i[0], jnp.int32)


_tc_scan = pl.pallas_call(
    _tc_scan_body,
    grid=(_TC_STEPS,),
    in_specs=[
        pl.BlockSpec((_INFEATURES,), lambda i: (0,)),
        pl.BlockSpec((_TC_BLOCK, _INFEATURES),
                     lambda i: (i + _SC_ROWS // _TC_BLOCK, 0)),
    ],
    out_specs=[
        pl.BlockSpec((1, 128), lambda i: (0, 0)),
        pl.BlockSpec((1, 128), lambda i: (0, 0)),
        pl.BlockSpec((1, 128), lambda i: (0, 0)),
    ],
    out_shape=[
        jax.ShapeDtypeStruct((1, 128), jnp.float32),
        jax.ShapeDtypeStruct((1, 128), jnp.float32),
        jax.ShapeDtypeStruct((1, 128), jnp.int32),
    ],
    scratch_shapes=[
        pltpu.SMEM((4,), jnp.float32),
        pltpu.SMEM((2,), jnp.int32),
    ],
)


def _merge_body(x_ref, sd_ref, sn_ref, si_ref, td_ref, tn_ref, ti_ref,
                out_ref):
    x = x_ref[...]
    xx = jnp.sum(x * x)
    big_i = jnp.int32(2**31 - 1)

    sd = sd_ref[...]
    sn = sn_ref[...]
    si = si_ref[...]
    gs = sd * jnp.abs(sd) / jnp.maximum(sn, 1e-16)

    td = jnp.max(td_ref[...])
    tn = jnp.max(tn_ref[...])
    ti = jnp.max(ti_ref[...])
    gt = td * jnp.abs(td) / jnp.maximum(tn, 1e-16)

    gm = jnp.maximum(jnp.max(gs), gt)
    bi = jnp.minimum(jnp.min(jnp.where(gs == gm, si, big_i)),
                     jnp.where(gt == gm, ti, big_i))
    in_sc = jnp.min(jnp.where(gs == gm, si, big_i)) <= bi
    bd = jnp.where(bi == ti, td, jnp.max(jnp.where(si == bi, sd, -3.4e38)))
    bn = jnp.where(bi == ti, tn, jnp.max(jnp.where(si == bi, sn, -3.4e38)))
    bd = jnp.where(in_sc, jnp.max(jnp.where(si == bi, sd, -3.4e38)), bd)
    bn = jnp.where(in_sc, jnp.max(jnp.where(si == bi, sn, -3.4e38)), bn)
    val = bd / (jnp.maximum(jnp.sqrt(bn), _EPS) * jnp.maximum(jnp.sqrt(xx), _EPS))
    fi = lax.broadcasted_iota(jnp.int32, (_CAPACITY,), 0)
    out_ref[...] = jnp.where(fi == bi, val, 0.0)


_merge = pl.pallas_call(
    _merge_body,
    out_shape=jax.ShapeDtypeStruct((_CAPACITY,), jnp.float32),
)


@jax.jit
def kernel(x, memory):
    sd, sn, si = _build_phase1()(x, memory)
    td, tn, ti = _tc_scan(x, memory)
    return _merge(x, sd, sn, si, td, tn, ti)
